# Initial kernel scaffold; baseline (speedup 1.0000x reference)
import jax
import jax.numpy as jnp
from jax.experimental import pallas as pl


def kernel(x):
    raise NotImplementedError("write your pallas kernel here")


# speedup vs baseline: 30.3690x; 30.3690x over previous
import functools

import jax
import jax.numpy as jnp
from jax import lax
from jax.experimental import pallas as pl
from jax.experimental.pallas import tpu as pltpu
from jax.experimental.pallas import tpu_sc as plsc

N = 8388608
NC = 2
NS = 16
NW = NC * NS
PER_W = N // NW
CHUNK = 32768
NCHUNK = PER_W // CHUNK
L = 16
GROUP = 8
NG = CHUNK // (L * GROUP)
CAP = 1024
TOPK = 64
K = 50
NEG_INF = float("-inf")
IMAX = 2**31 - 1


def _iota():
    return lax.iota(jnp.int32, L)


def _splat_f(v):
    return jnp.full((L,), v, jnp.float32)


def _splat_i(v):
    return jnp.full((L,), v, jnp.int32)


def _compact(cv, ci, tv, ti, cnt_s, tval_s):
    iota = _iota()

    def round_body(t, _):
        def scan_body(j, carry):
            bv, bi, bs = carry
            v = cv[pl.ds(j * L, L)]
            ivec = ci[pl.ds(j * L, L)]
            slot = iota + j * L
            better = (v > bv) | ((v == bv) & (ivec < bi))
            return (jnp.where(better, v, bv),
                    jnp.where(better, ivec, bi),
                    jnp.where(better, slot, bs))

        bv, bi, bs = lax.fori_loop(
            0, CAP // L, scan_body,
            (_splat_f(NEG_INF), _splat_i(IMAX), iota))
        mv = jnp.max(bv)
        mi = jnp.min(jnp.where(bv == mv, bi, IMAX))
        sel = (bv == mv) & (bi == mi)
        slot_sel = jnp.min(jnp.where(sel, bs, IMAX))
        lane0 = iota == 0
        plsc.store_scatter(tv, [_splat_i(t)], _splat_f(0) + mv, mask=lane0)
        plsc.store_scatter(ti, [_splat_i(t)], _splat_i(0) + mi, mask=lane0)
        plsc.store_scatter(cv, [_splat_i(0) + slot_sel], _splat_f(NEG_INF),
                           mask=lane0)
        plsc.store_scatter(ci, [_splat_i(0) + slot_sel], _splat_i(IMAX),
                           mask=lane0)
        return 0

    lax.fori_loop(0, TOPK, round_body, 0)

    def clear_body(j, _):
        cv[pl.ds(j * L, L)] = _splat_f(NEG_INF)
        ci[pl.ds(j * L, L)] = _splat_i(IMAX)
        return 0

    lax.fori_loop(0, CAP // L, clear_body, 0)
    for j in range(TOPK // L):
        cv[pl.ds(j * L, L)] = tv[pl.ds(j * L, L)]
        ci[pl.ds(j * L, L)] = ti[pl.ds(j * L, L)]
    cnt_s[0] = TOPK
    tval_s[0] = jnp.min(tv[pl.ds(TOPK - L, L)])


def _phase1_body(x_hbm, cv_out, ci_out, buf, cv, ci, tv, ti, cnt_s, tval_s):
    iota = _iota()
    wid = lax.axis_index("s") * NC + lax.axis_index("c")
    base = wid * PER_W

    def init_body(j, _):
        cv[pl.ds(j * L, L)] = _splat_f(NEG_INF)
        ci[pl.ds(j * L, L)] = _splat_i(IMAX)
        return 0

    lax.fori_loop(0, CAP // L, init_body, 0)
    cnt_s[0] = 0
    tval_s[0] = NEG_INF

    def chunk_body(c, _):
        origin = base + c * CHUNK
        pltpu.sync_copy(x_hbm.at[pl.ds(origin, CHUNK)], buf)

        def group_body(g, _):
            gbase = g * (L * GROUP)
            tval = tval_s[0]
            vs = [buf[pl.ds(gbase + k * L, L)] for k in range(GROUP)]
            acc = vs[0]
            for k in range(1, GROUP):
                acc = jnp.maximum(acc, vs[k])
            mx = jnp.max(acc)

            @pl.when(mx >= tval)
            def _():
                for k in range(GROUP):
                    v = vs[k]
                    m = v >= tval
                    csum = jnp.cumsum(m.astype(jnp.int32))
                    pos = cnt_s[0] + csum - 1
                    idxv = iota + (origin + gbase + k * L)
                    plsc.store_scatter(cv, [pos], v, mask=m)
                    plsc.store_scatter(ci, [pos], idxv, mask=m)
                    cnt_s[0] = cnt_s[0] + jnp.max(csum)

                @pl.when(cnt_s[0] > CAP - L * GROUP)
                def _():
                    _compact(cv, ci, tv, ti, cnt_s, tval_s)

            return 0

        lax.fori_loop(0, NG, group_body, 0)
        return 0

    lax.fori_loop(0, NCHUNK, chunk_body, 0)
    _compact(cv, ci, tv, ti, cnt_s, tval_s)
    pltpu.sync_copy(tv, cv_out.at[pl.ds(wid * TOPK, TOPK)])
    pltpu.sync_copy(ti, ci_out.at[pl.ds(wid * TOPK, TOPK)])


def _phase2_body(cv_hbm, ci_hbm, y_hbm, mv, mi, win, zbuf):
    iota = _iota()
    wid = lax.axis_index("s") * NC + lax.axis_index("c")
    base = wid * PER_W
    total = NW * TOPK
    pltpu.sync_copy(cv_hbm, mv)
    pltpu.sync_copy(ci_hbm, mi)

    def zero_body(j, _):
        zbuf[pl.ds(j * L, L)] = _splat_f(0.0)
        return 0

    lax.fori_loop(0, CHUNK // L, zero_body, 0)
    for j in range(TOPK // L):
        win[pl.ds(j * L, L)] = _splat_i(-1)

    lane0 = iota == 0

    def round_body(t, _):
        def scan_body(j, carry):
            bv, bi, bs = carry
            v = mv[pl.ds(j * L, L)]
            ivec = mi[pl.ds(j * L, L)]
            slot = iota + j * L
            better = (v > bv) | ((v == bv) & (ivec < bi))
            return (jnp.where(better, v, bv),
                    jnp.where(better, ivec, bi),
                    jnp.where(better, slot, bs))

        bv, bi, bs = lax.fori_loop(
            0, total // L, scan_body,
            (_splat_f(NEG_INF), _splat_i(IMAX), iota))
        mvx = jnp.max(bv)
        mix = jnp.min(jnp.where(bv == mvx, bi, IMAX))
        sel = (bv == mvx) & (bi == mix)
        slot_sel = jnp.min(jnp.where(sel, bs, IMAX))
        plsc.store_scatter(win, [_splat_i(t)], _splat_i(0) + mix, mask=lane0)
        plsc.store_scatter(mv, [_splat_i(0) + slot_sel], _splat_f(NEG_INF),
                           mask=lane0)
        plsc.store_scatter(mi, [_splat_i(0) + slot_sel], _splat_i(IMAX),
                           mask=lane0)
        return 0

    lax.fori_loop(0, K, round_body, 0)

    ws = [win[pl.ds(j * L, L)] for j in range(TOPK // L)]

    def chunk_body(c, _):
        lo = base + c * CHUNK
        masks = [(w >= lo) & (w < lo + CHUNK) for w in ws]
        anym = masks[0]
        for m in masks[1:]:
            anym = anym | m
        has = jnp.max(anym.astype(jnp.int32))

        @pl.when(has > 0)
        def _():
            for w, m in zip(ws, masks):
                loc = jnp.where(m, w - lo, 0)
                plsc.store_scatter(zbuf, [loc], _splat_f(1.0), mask=m)

        pltpu.sync_copy(zbuf, y_hbm.at[pl.ds(lo, CHUNK)])

        @pl.when(has > 0)
        def _():
            for w, m in zip(ws, masks):
                loc = jnp.where(m, w - lo, 0)
                plsc.store_scatter(zbuf, [loc], _splat_f(0.0), mask=m)

        return 0

    lax.fori_loop(0, NCHUNK, chunk_body, 0)


def _make_phase1():
    mesh = plsc.VectorSubcoreMesh(core_axis_name="c", subcore_axis_name="s")
    return functools.partial(
        pl.kernel,
        mesh=mesh,
        compiler_params=pltpu.CompilerParams(needs_layout_passes=False),
        out_type=[
            jax.ShapeDtypeStruct((NW * TOPK,), jnp.float32),
            jax.ShapeDtypeStruct((NW * TOPK,), jnp.int32),
        ],
        scratch_types=[
            pltpu.VMEM((CHUNK,), jnp.float32),
            pltpu.VMEM((CAP,), jnp.float32),
            pltpu.VMEM((CAP,), jnp.int32),
            pltpu.VMEM((TOPK,), jnp.float32),
            pltpu.VMEM((TOPK,), jnp.int32),
            pltpu.SMEM((1,), jnp.int32),
            pltpu.SMEM((1,), jnp.float32),
        ],
    )(_phase1_body)


def _make_phase2():
    mesh = plsc.VectorSubcoreMesh(core_axis_name="c", subcore_axis_name="s")
    return functools.partial(
        pl.kernel,
        mesh=mesh,
        compiler_params=pltpu.CompilerParams(needs_layout_passes=False),
        out_type=jax.ShapeDtypeStruct((N,), jnp.float32),
        scratch_types=[
            pltpu.VMEM((NW * TOPK,), jnp.float32),
            pltpu.VMEM((NW * TOPK,), jnp.int32),
            pltpu.VMEM((TOPK,), jnp.int32),
            pltpu.VMEM((CHUNK,), jnp.float32),
        ],
    )(_phase2_body)


_phase1 = _make_phase1()
_phase2 = _make_phase2()


def kernel(x):
    cand_v, cand_i = _phase1(x)
    return _phase2(cand_v, cand_i)
